# baseline (speedup 1.0000x reference)
import functools

import jax
import jax.numpy as jnp
import numpy as np
from jax import lax
from jax.experimental import pallas as pl
from jax.experimental.pallas import tpu as pltpu
from jax.experimental.pallas import tpu_sc as plsc

N = 10000
E = 320000
D = 128
DE = 128
S = 4
FC = 64
SH = 1
IN1 = D + D + DE

NC = 2
NS = 16
NW = NC * NS
EPT = E // NW
B = 120
NBPT = (EPT + B - 1) // B
CHUNK = 80
NCHUNK = N // CHUNK

_INV_S = float(1.0 / np.sqrt(IN1 * SH))
_SC_SCALE = float(1.0 / np.sqrt(D * S))

_QW = np.empty(32, np.int64)
_QW[0::2] = np.arange(16)
_QW[1::2] = 16 + np.arange(16)
_Q = np.concatenate([g * 32 + _QW for g in range(D // 32)])


def _node_prep_body(nf_ref, oh_ref, wpre_ref, bpre_ref, wa_ref, wb_ref,
                    wsc_ref, u_ref, v_ref, sc_ref):
    nf = nf_ref[...]
    x = jnp.dot(nf, wpre_ref[...], preferred_element_type=jnp.float32) + bpre_ref[...]
    u_ref[...] = jnp.dot(x, wa_ref[...], preferred_element_type=jnp.float32) * _INV_S
    v_ref[...] = jnp.dot(x, wb_ref[...], preferred_element_type=jnp.float32) * _INV_S
    oh = oh_ref[...]
    acc = jnp.zeros_like(nf)
    for s in range(S):
        acc += jnp.dot(nf * oh[:, s:s + 1], wsc_ref[s],
                       preferred_element_type=jnp.float32)
    sc_ref[...] = acc * _SC_SCALE


def _node_prep(node_fea, node_one_hot, w_pre, b_pre, wa, wb, wsc_t):
    blk = 2000
    grid = N // blk
    return pl.pallas_call(
        _node_prep_body,
        grid=(grid,),
        in_specs=[
            pl.BlockSpec((blk, D), lambda i: (i, 0)),
            pl.BlockSpec((blk, S), lambda i: (i, 0)),
            pl.BlockSpec((D, D), lambda i: (0, 0)),
            pl.BlockSpec((1, D), lambda i: (0, 0)),
            pl.BlockSpec((D, D), lambda i: (0, 0)),
            pl.BlockSpec((D, D), lambda i: (0, 0)),
            pl.BlockSpec((S, D, D), lambda i: (0, 0, 0)),
        ],
        out_specs=[
            pl.BlockSpec((blk, D), lambda i: (i, 0)),
            pl.BlockSpec((blk, D), lambda i: (i, 0)),
            pl.BlockSpec((blk, D), lambda i: (i, 0)),
        ],
        out_shape=[
            jax.ShapeDtypeStruct((N, D), jnp.float32),
            jax.ShapeDtypeStruct((N, D), jnp.float32),
            jax.ShapeDtypeStruct((N, D), jnp.float32),
        ],
    )(node_fea, node_one_hot, w_pre, b_pre.reshape(1, D), wa, wb, wsc_t)


def _edge_prep_body(ef_ref, ele_ref, wc_ref, w1_ref, b1_ref, w2_ref,
                    b2_ref, w3_ref, b3_ref, cw_ref):
    cw_ref[:, :D] = (jnp.dot(ef_ref[...], wc_ref[...],
                             preferred_element_type=jnp.float32)
                     * _INV_S).astype(jnp.bfloat16)
    h = jax.nn.silu(jnp.dot(ele_ref[...], w1_ref[...],
                            preferred_element_type=jnp.float32) + b1_ref[...])
    h = jax.nn.silu(jnp.dot(h, w2_ref[...],
                            preferred_element_type=jnp.float32) + b2_ref[...])
    cw_ref[:, D:] = (jnp.dot(h, w3_ref[...],
                             preferred_element_type=jnp.float32)
                     + b3_ref[...]).astype(jnp.bfloat16)


def _edge_prep(edge_fea, ele, wc, w1, b1, w2, b2, w3, b3):
    blk = 4000
    grid = E // blk
    return pl.pallas_call(
        _edge_prep_body,
        grid=(grid,),
        in_specs=[
            pl.BlockSpec((blk, DE), lambda i: (i, 0)),
            pl.BlockSpec((blk, FC), lambda i: (i, 0)),
            pl.BlockSpec((DE, D), lambda i: (0, 0)),
            pl.BlockSpec((FC, 64), lambda i: (0, 0)),
            pl.BlockSpec((1, 64), lambda i: (0, 0)),
            pl.BlockSpec((64, 64), lambda i: (0, 0)),
            pl.BlockSpec((1, 64), lambda i: (0, 0)),
            pl.BlockSpec((64, D), lambda i: (0, 0)),
            pl.BlockSpec((1, D), lambda i: (0, 0)),
        ],
        out_specs=pl.BlockSpec((blk, 2 * D), lambda i: (i, 0)),
        out_shape=jax.ShapeDtypeStruct((E, 2 * D), jnp.bfloat16),
    )(edge_fea, ele, wc, w1, b1.reshape(1, 64), w2, b2.reshape(1, 64),
      w3, b3.reshape(1, D))


def _sc_body(u_hbm, v_hbm, idx_hbm, cw_hbm, sh_hbm, out_hbm,
             idxb, gi, gj, cwb, shs, sem1, sem2, acc):
    cid = lax.axis_index("c")
    sid = lax.axis_index("s")
    zeros16 = jnp.zeros((16,), jnp.float32)
    cmask = jnp.int32(-65536)

    wid = cid * NS + sid

    def _zero_row(r, _):
        for q in range(D // 16):
            gi[r, pl.ds(q * 16, 16)] = zeros16
        return 0
    lax.fori_loop(0, CHUNK, _zero_row, 0)

    def _zero_chunk(k, _):
        chunk = sid + NS * k
        @pl.when(chunk < NCHUNK)
        def _():
            pltpu.sync_copy(gi.at[pl.ds(0, CHUNK)],
                            acc.at[pl.ds(chunk * CHUNK, CHUNK)])
        return 0
    lax.fori_loop(0, (NCHUNK + NS - 1) // NS, _zero_chunk, 0)

    plsc.subcore_barrier()

    def _block(k, _):
        bk = wid * NBPT + k
        est = wid * EPT + jnp.minimum(k * B, EPT - B)
        pltpu.sync_copy(idx_hbm.at[bk], idxb)
        d1 = pltpu.async_copy(u_hbm.at[idxb.at[0]], gi, sem1)
        d2 = pltpu.async_copy(v_hbm.at[idxb.at[1]], gj, sem2)
        @pl.when((k & 1) == 0)
        def _():
            pltpu.sync_copy(sh_hbm.at[wid, k // 2], shs)
        pltpu.sync_copy(cw_hbm.at[pl.ds(est, B)], cwb)
        d1.wait()
        d2.wait()

        p2 = k & 1

        def _row(r, _):
            sv = shs[pl.ds(p2 * (B * 16) + r * 16, 16)]
            for q2 in range(D // 32):
                cc = cwb[r, pl.ds(q2 * 16, 16)]
                ww = cwb[r, pl.ds(64 + q2 * 16, 16)]
                for h in range(2):
                    if h == 0:
                        ch = lax.bitcast_convert_type(lax.shift_left(cc, 16), jnp.float32)
                        wh = lax.bitcast_convert_type(lax.shift_left(ww, 16), jnp.float32)
                    else:
                        ch = lax.bitcast_convert_type(lax.bitwise_and(cc, cmask), jnp.float32)
                        wh = lax.bitcast_convert_type(lax.bitwise_and(ww, cmask), jnp.float32)
                    sl = pl.ds(q2 * 32 + 16 * h, 16)
                    z = (gi[r, sl] + gj[r, sl] + ch) * sv
                    m = z / (1.0 + jnp.exp(-z)) * wh
                    gi[r, sl] = m
            return 0
        lax.fori_loop(0, B, _row, 0)

        pltpu.sync_copy(gi, acc.at[idxb.at[0]], add=True)
        return 0
    lax.fori_loop(0, NBPT, _block, 0)

    plsc.subcore_barrier()

    def _dump_chunk(k, _):
        chunk = sid + NS * k
        @pl.when(chunk < NCHUNK)
        def _():
            pltpu.sync_copy(acc.at[pl.ds(chunk * CHUNK, CHUNK)],
                            out_hbm.at[pl.ds(cid * N + chunk * CHUNK, CHUNK)])
        return 0
    lax.fori_loop(0, (NCHUNK + NS - 1) // NS, _dump_chunk, 0)


def _sc_aggregate(u, v, idx3, cw_i32, shblk):
    mesh = plsc.VectorSubcoreMesh(core_axis_name="c", subcore_axis_name="s",
                                  num_cores=NC, num_subcores=NS)
    f = pl.kernel(
        _sc_body,
        out_type=jax.ShapeDtypeStruct((NC * N, D), jnp.float32),
        mesh=mesh,
        scratch_types=[
            pltpu.VMEM((2, B), jnp.int32),
            pltpu.VMEM((B, D), jnp.float32),
            pltpu.VMEM((B, D), jnp.float32),
            pltpu.VMEM((B, D), jnp.int32),
            pltpu.VMEM((2 * B * 16,), jnp.float32),
            pltpu.SemaphoreType.DMA,
            pltpu.SemaphoreType.DMA,
            pltpu.VMEM_SHARED((N, D), jnp.float32),
        ],
    )
    return f(u, v, idx3, cw_i32, shblk)


def _epilogue_body(aggp_ref, sc_ref, nf_ref, wpost_ref, bpost_ref,
                   gamma_ref, beta_ref, out_ref):
    agg = aggp_ref[0] + aggp_ref[1]
    o = jnp.dot(agg, wpost_ref[...], preferred_element_type=jnp.float32)
    o = o + bpost_ref[...] + sc_ref[...]
    m_d = jnp.mean(o, axis=0, keepdims=True)
    s_d = jnp.mean(o * o, axis=0, keepdims=True)
    rms = jnp.mean(s_d - m_d * m_d)
    inv = lax.rsqrt(rms + 1e-5)
    out_ref[...] = ((o - m_d) * inv * gamma_ref[...] + beta_ref[...]
                    + nf_ref[...])


def _epilogue(aggp, sc, node_fea, w_post, b_post, gamma, beta):
    return pl.pallas_call(
        _epilogue_body,
        out_shape=jax.ShapeDtypeStruct((N, D), jnp.float32),
    )(aggp.reshape(NC, N, D), sc, node_fea, w_post, b_post.reshape(1, D),
      gamma.reshape(1, D), beta.reshape(1, D))


def kernel(node_fea, node_one_hot, edge_sh, edge_fea, edge_length_embedded,
           edge_index, batch, selfloop_edge, edge_length,
           W_pre, b_pre, W_tp, W1, b1, W2, b2, W3, b3, W_post, b_post,
           W_sc, gamma, beta):
    w_flat = W_tp.reshape(IN1, D)
    wa = w_flat[:D]
    wb = w_flat[D:2 * D]
    wc = w_flat[2 * D:][:, _Q]
    w3 = W3[:, _Q]
    b3 = b3[_Q]
    wsc_t = W_sc.transpose(1, 0, 2)

    u, v, sc = _node_prep(node_fea, node_one_hot, W_pre, b_pre, wa, wb, wsc_t)
    cw = _edge_prep(edge_fea, edge_length_embedded, wc, W1, b1, W2, b2, w3, b3)
    cw_i32 = lax.bitcast_convert_type(cw.reshape(E, D, 2), jnp.int32)

    ii = edge_index[0].astype(jnp.int32)
    jj = edge_index[1].astype(jnp.int32)
    sh = edge_sh.reshape(E)

    ks = jnp.minimum(jnp.arange(NBPT, dtype=jnp.int32) * B, EPT - B)
    pos = (jnp.arange(NW, dtype=jnp.int32)[:, None, None] * EPT
           + ks[None, :, None]
           + jnp.arange(B, dtype=jnp.int32)[None, None, :])
    fresh = (ks[:, None] + jnp.arange(B, dtype=jnp.int32)[None, :]
             >= jnp.arange(NBPT, dtype=jnp.int32)[:, None] * B)
    posf = pos.reshape(NW * NBPT, B)
    idx3 = jnp.stack([jnp.take(ii, posf, axis=0),
                      jnp.take(jj, posf, axis=0)], axis=1)
    shblk = jnp.take(sh, pos, axis=0) * fresh[None, :, :]
    shblk = jnp.broadcast_to(shblk[..., None], (NW, NBPT, B, 16))
    shblk = shblk.reshape(NW, NBPT // 2, 2 * B * 16)

    aggp = _sc_aggregate(u, v, idx3, cw_i32, shblk)
    return _epilogue(aggp, sc, node_fea, W_post, b_post, gamma, beta)


# speedup vs baseline: 1.5457x; 1.5457x over previous
import functools

import jax
import jax.numpy as jnp
import numpy as np
from jax import lax
from jax.experimental import pallas as pl
from jax.experimental.pallas import tpu as pltpu
from jax.experimental.pallas import tpu_sc as plsc

N = 10000
E = 320000
D = 128
DE = 128
S = 4
FC = 64
SH = 1
IN1 = D + D + DE

NC = 2
NS = 16
NW = NC * NS
EPT = E // NW
B = 120
NBPT = (EPT + B - 1) // B
CHUNK = 80
NCHUNK = N // CHUNK

_INV_S = float(1.0 / np.sqrt(IN1 * SH))
_SC_SCALE = float(1.0 / np.sqrt(D * S))

_Q2 = np.concatenate(
    [np.concatenate([g * 32 + np.arange(16) for g in range(D // 32)]),
     np.concatenate([g * 32 + 16 + np.arange(16) for g in range(D // 32)])])


def _node_prep_body(nf_ref, oh_ref, wpre_ref, bpre_ref, wa_ref, wb_ref,
                    wsc_ref, u_ref, v_ref, sc_ref):
    nf = nf_ref[...]
    x = jnp.dot(nf, wpre_ref[...], preferred_element_type=jnp.float32) + bpre_ref[...]
    u_ref[...] = jnp.dot(x, wa_ref[...], preferred_element_type=jnp.float32) * _INV_S
    v_ref[...] = jnp.dot(x, wb_ref[...], preferred_element_type=jnp.float32) * _INV_S
    oh = oh_ref[...]
    acc = jnp.zeros_like(nf)
    for s in range(S):
        acc += jnp.dot(nf * oh[:, s:s + 1], wsc_ref[s],
                       preferred_element_type=jnp.float32)
    sc_ref[...] = acc * _SC_SCALE


def _node_prep(node_fea, node_one_hot, w_pre, b_pre, wa, wb, wsc_t):
    blk = 2000
    grid = N // blk
    return pl.pallas_call(
        _node_prep_body,
        grid=(grid,),
        in_specs=[
            pl.BlockSpec((blk, D), lambda i: (i, 0)),
            pl.BlockSpec((blk, S), lambda i: (i, 0)),
            pl.BlockSpec((D, D), lambda i: (0, 0)),
            pl.BlockSpec((1, D), lambda i: (0, 0)),
            pl.BlockSpec((D, D), lambda i: (0, 0)),
            pl.BlockSpec((D, D), lambda i: (0, 0)),
            pl.BlockSpec((S, D, D), lambda i: (0, 0, 0)),
        ],
        out_specs=[
            pl.BlockSpec((blk, D), lambda i: (i, 0)),
            pl.BlockSpec((blk, D), lambda i: (i, 0)),
            pl.BlockSpec((blk, D), lambda i: (i, 0)),
        ],
        out_shape=[
            jax.ShapeDtypeStruct((N, D), jnp.float32),
            jax.ShapeDtypeStruct((N, D), jnp.float32),
            jax.ShapeDtypeStruct((N, D), jnp.float32),
        ],
    )(node_fea, node_one_hot, w_pre, b_pre.reshape(1, D), wa, wb, wsc_t)


def _bf16_pack_words(x):
    bits = lax.bitcast_convert_type(
        x.astype(jnp.bfloat16).astype(jnp.float32), jnp.int32)
    lo = lax.shift_right_logical(bits[:, :64], 16)
    hi = jnp.bitwise_and(bits[:, 64:], jnp.int32(-65536))
    return jnp.bitwise_or(lo, hi)


def _edge_prep_body(ef_ref, ele_ref, wc_ref, w1_ref, b1_ref, w2_ref,
                    b2_ref, w3_ref, b3_ref, cw_ref):
    c = jnp.dot(ef_ref[...], wc_ref[...],
                preferred_element_type=jnp.float32) * _INV_S
    cw_ref[:, :D // 2] = _bf16_pack_words(c)
    h = jax.nn.silu(jnp.dot(ele_ref[...], w1_ref[...],
                            preferred_element_type=jnp.float32) + b1_ref[...])
    h = jax.nn.silu(jnp.dot(h, w2_ref[...],
                            preferred_element_type=jnp.float32) + b2_ref[...])
    w = jnp.dot(h, w3_ref[...],
                preferred_element_type=jnp.float32) + b3_ref[...]
    cw_ref[:, D // 2:] = _bf16_pack_words(w)


def _edge_prep(edge_fea, ele, wc, w1, b1, w2, b2, w3, b3):
    blk = 4000
    grid = E // blk
    return pl.pallas_call(
        _edge_prep_body,
        grid=(grid,),
        in_specs=[
            pl.BlockSpec((blk, DE), lambda i: (i, 0)),
            pl.BlockSpec((blk, FC), lambda i: (i, 0)),
            pl.BlockSpec((DE, D), lambda i: (0, 0)),
            pl.BlockSpec((FC, 64), lambda i: (0, 0)),
            pl.BlockSpec((1, 64), lambda i: (0, 0)),
            pl.BlockSpec((64, 64), lambda i: (0, 0)),
            pl.BlockSpec((1, 64), lambda i: (0, 0)),
            pl.BlockSpec((64, D), lambda i: (0, 0)),
            pl.BlockSpec((1, D), lambda i: (0, 0)),
        ],
        out_specs=pl.BlockSpec((blk, D), lambda i: (i, 0)),
        out_shape=jax.ShapeDtypeStruct((E, D), jnp.int32),
    )(edge_fea, ele, wc, w1, b1.reshape(1, 64), w2, b2.reshape(1, 64),
      w3, b3.reshape(1, D))


def _sc_body(u_hbm, v_hbm, idx_hbm, cw_hbm, sh_hbm, out_hbm,
             idxb, gi, gj, cwb, shs, sem1, sem2, acc):
    cid = lax.axis_index("c")
    sid = lax.axis_index("s")
    zeros16 = jnp.zeros((16,), jnp.float32)
    cmask = jnp.int32(-65536)

    wid = cid * NS + sid

    def _zero_row(r, _):
        for q in range(D // 16):
            gi[r, pl.ds(q * 16, 16)] = zeros16
        return 0
    lax.fori_loop(0, CHUNK, _zero_row, 0)

    def _zero_chunk(k, _):
        chunk = sid + NS * k
        @pl.when(chunk < NCHUNK)
        def _():
            pltpu.sync_copy(gi.at[pl.ds(0, CHUNK)],
                            acc.at[pl.ds(chunk * CHUNK, CHUNK)])
        return 0
    lax.fori_loop(0, (NCHUNK + NS - 1) // NS, _zero_chunk, 0)

    plsc.subcore_barrier()

    def _block(k, _):
        bk = wid * NBPT + k
        est = wid * EPT + jnp.minimum(k * B, EPT - B)
        pltpu.sync_copy(idx_hbm.at[bk], idxb)
        d1 = pltpu.async_copy(u_hbm.at[idxb.at[0]], gi, sem1)
        d2 = pltpu.async_copy(v_hbm.at[idxb.at[1]], gj, sem2)
        @pl.when((k & 1) == 0)
        def _():
            pltpu.sync_copy(
                sh_hbm.at[pl.ds((wid * NBPT + k) * (B * 16), 2 * B * 16)],
                shs)
        pltpu.sync_copy(cw_hbm.at[pl.ds(est, B)], cwb)
        d1.wait()
        d2.wait()

        p2 = k & 1

        def _row(r, _):
            sv = shs[pl.ds(p2 * (B * 16) + r * 16, 16)]
            for q2 in range(D // 32):
                cc = cwb[r, pl.ds(q2 * 16, 16)]
                ww = cwb[r, pl.ds(64 + q2 * 16, 16)]
                for h in range(2):
                    if h == 0:
                        ch = lax.bitcast_convert_type(lax.shift_left(cc, 16), jnp.float32)
                        wh = lax.bitcast_convert_type(lax.shift_left(ww, 16), jnp.float32)
                    else:
                        ch = lax.bitcast_convert_type(lax.bitwise_and(cc, cmask), jnp.float32)
                        wh = lax.bitcast_convert_type(lax.bitwise_and(ww, cmask), jnp.float32)
                    sl = pl.ds(q2 * 32 + 16 * h, 16)
                    z = (gi[r, sl] + gj[r, sl] + ch) * sv
                    m = z / (1.0 + jnp.exp(-z)) * wh
                    gi[r, sl] = m
            return 0
        lax.fori_loop(0, B, _row, 0)

        pltpu.sync_copy(gi, acc.at[idxb.at[0]], add=True)
        return 0
    lax.fori_loop(0, NBPT, _block, 0)

    plsc.subcore_barrier()

    def _dump_chunk(k, _):
        chunk = sid + NS * k
        @pl.when(chunk < NCHUNK)
        def _():
            pltpu.sync_copy(acc.at[pl.ds(chunk * CHUNK, CHUNK)],
                            out_hbm.at[pl.ds(cid * N + chunk * CHUNK, CHUNK)])
        return 0
    lax.fori_loop(0, (NCHUNK + NS - 1) // NS, _dump_chunk, 0)


def _sc_aggregate(u, v, idx3, cw_i32, shblk):
    mesh = plsc.VectorSubcoreMesh(core_axis_name="c", subcore_axis_name="s",
                                  num_cores=NC, num_subcores=NS)
    f = pl.kernel(
        _sc_body,
        out_type=jax.ShapeDtypeStruct((NC * N, D), jnp.float32),
        mesh=mesh,
        scratch_types=[
            pltpu.VMEM((2, B), jnp.int32),
            pltpu.VMEM((B, D), jnp.float32),
            pltpu.VMEM((B, D), jnp.float32),
            pltpu.VMEM((B, D), jnp.int32),
            pltpu.VMEM((2 * B * 16,), jnp.float32),
            pltpu.SemaphoreType.DMA,
            pltpu.SemaphoreType.DMA,
            pltpu.VMEM_SHARED((N, D), jnp.float32),
        ],
    )
    return f(u, v, idx3, cw_i32, shblk)


def _epilogue_body(aggp_ref, sc_ref, nf_ref, wpost_ref, bpost_ref,
                   gamma_ref, beta_ref, out_ref):
    agg = aggp_ref[0] + aggp_ref[1]
    o = jnp.dot(agg, wpost_ref[...], preferred_element_type=jnp.float32)
    o = o + bpost_ref[...] + sc_ref[...]
    m_d = jnp.mean(o, axis=0, keepdims=True)
    s_d = jnp.mean(o * o, axis=0, keepdims=True)
    rms = jnp.mean(s_d - m_d * m_d)
    inv = lax.rsqrt(rms + 1e-5)
    out_ref[...] = ((o - m_d) * inv * gamma_ref[...] + beta_ref[...]
                    + nf_ref[...])


def _epilogue(aggp, sc, node_fea, w_post, b_post, gamma, beta):
    return pl.pallas_call(
        _epilogue_body,
        out_shape=jax.ShapeDtypeStruct((N, D), jnp.float32),
    )(aggp.reshape(NC, N, D), sc, node_fea, w_post, b_post.reshape(1, D),
      gamma.reshape(1, D), beta.reshape(1, D))


def kernel(node_fea, node_one_hot, edge_sh, edge_fea, edge_length_embedded,
           edge_index, batch, selfloop_edge, edge_length,
           W_pre, b_pre, W_tp, W1, b1, W2, b2, W3, b3, W_post, b_post,
           W_sc, gamma, beta):
    w_flat = W_tp.reshape(IN1, D)
    wa = w_flat[:D]
    wb = w_flat[D:2 * D]
    wc = w_flat[2 * D:][:, _Q2]
    w3 = W3[:, _Q2]
    b3 = b3[_Q2]
    wsc_t = W_sc.transpose(1, 0, 2)

    u, v, sc = _node_prep(node_fea, node_one_hot, W_pre, b_pre, wa, wb, wsc_t)
    cw_i32 = _edge_prep(edge_fea, edge_length_embedded, wc, W1, b1, W2, b2,
                        w3, b3)

    ii = edge_index[0].astype(jnp.int32)
    jj = edge_index[1].astype(jnp.int32)
    sh = edge_sh.reshape(E)

    def blocks_of(x):
        xw = x.reshape(NW, EPT)
        core = xw[:, :(NBPT - 1) * B].reshape(NW, NBPT - 1, B)
        tail = xw[:, EPT - B:].reshape(NW, 1, B)
        return jnp.concatenate([core, tail], axis=1)

    fresh = np.ones((NBPT, B), np.float32)
    fresh[-1, :NBPT * B - EPT] = 0.0
    idx3 = jnp.stack([blocks_of(ii), blocks_of(jj)],
                     axis=2).reshape(NW * NBPT, 2, B)
    shblk = blocks_of(sh) * jnp.asarray(fresh)[None, :, :]
    shblk = jnp.broadcast_to(shblk[..., None], (NW, NBPT, B, 16))
    shblk = shblk.reshape(NW * NBPT * B * 16)

    aggp = _sc_aggregate(u, v, idx3, cw_i32, shblk)
    return _epilogue(aggp, sc, node_fea, W_post, b_post, gamma, beta)
